# baseline (speedup 1.0000x reference)
import functools

import jax
import jax.numpy as jnp
from jax import lax
from jax.experimental import pallas as pl
from jax.experimental.pallas import tpu as pltpu
from jax.experimental.pallas import tpu_sc as plsc

N = 10000
E = 320000
IN = 128
EMB = 32
NT = 512
OUT = 1024
H1 = 512
NC = 10
B = 64

R = 400
G = N // R
F = 8
W = 128
NBT = 160
GS = 16
NGRP = NBT // GS
EP = 16 * NBT * 128
ZROWS = 632
SLAB = 16 * ZROWS
FL_LAST = N - 15 * ZROWS


def _k1_body(x_ref, embp_ref, wg0p_ref, hs_ref, *m_refs):
    xv = x_ref[...]
    ids = xv[:, 0:1].astype(jnp.int32)
    iota = lax.broadcasted_iota(jnp.int32, (R, NT), 1)
    oh = (ids == iota).astype(jnp.float32)
    te128 = jnp.dot(oh, embp_ref[...], preferred_element_type=jnp.float32)
    hs = jnp.concatenate(
        [te128[:, :EMB], xv[:, 1:], jnp.zeros((R, 97), jnp.float32)], axis=1)
    hs_ref[...] = hs
    m0 = jnp.dot(hs, wg0p_ref[...], preferred_element_type=jnp.float32)
    for f in range(F):
        m_refs[f][...] = m0[:, f * W:(f + 1) * W]


def _k1(x, embp, wg0p):
    return pl.pallas_call(
        _k1_body,
        grid=(G,),
        in_specs=[
            pl.BlockSpec((R, IN), lambda r: (r, 0)),
            pl.BlockSpec((NT, 128), lambda r: (0, 0)),
            pl.BlockSpec((256, OUT), lambda r: (0, 0)),
        ],
        out_specs=[pl.BlockSpec((R, 256), lambda r: (r, 0))] + [
            pl.BlockSpec((R, W), lambda r: (r, 0)) for _ in range(F)],
        out_shape=[jax.ShapeDtypeStruct((N, 256), jnp.float32)] + [
            jax.ShapeDtypeStruct((N, W), jnp.float32) for _ in range(F)],
    )(x, embp, wg0p)


def _k2_body(h_ref, w_ref, b_ref, gh_ref):
    gh_ref[...] = b_ref[...] + jnp.dot(h_ref[...], w_ref[...],
                                       preferred_element_type=jnp.float32)


def _k2(h, wt, b2):
    k = h.shape[1]
    return pl.pallas_call(
        _k2_body,
        grid=(G,),
        in_specs=[
            pl.BlockSpec((R, k), lambda r: (r, 0)),
            pl.BlockSpec((k, 3 * OUT), lambda r: (0, 0)),
            pl.BlockSpec((1, 3 * OUT), lambda r: (0, 0)),
        ],
        out_specs=pl.BlockSpec((R, 3 * OUT), lambda r: (r, 0)),
        out_shape=jax.ShapeDtypeStruct((N, 3 * OUT), jnp.float32),
    )(h, wt, b2)


def _k3_body(hs_ref, *rest):
    agg_refs = rest[:F]
    gh_ref, wiht_ref, wg1_ref, bih_ref, h1_ref = rest[F:F + 5]
    m_refs = rest[F + 5:]
    hs = hs_ref[...]
    agg = jnp.concatenate([a[...] for a in agg_refs], axis=1)
    gi = bih_ref[...] + jnp.dot(agg, wiht_ref[...],
                                preferred_element_type=jnp.float32)
    gh = gh_ref[...]
    r = jax.nn.sigmoid(gi[:, :OUT] + gh[:, :OUT])
    z = jax.nn.sigmoid(gi[:, OUT:2 * OUT] + gh[:, OUT:2 * OUT])
    nn = jnp.tanh(gi[:, 2 * OUT:] + r * gh[:, 2 * OUT:])
    h0p = jnp.concatenate([hs, jnp.zeros((R, OUT - 256), jnp.float32)], axis=1)
    h1 = (1.0 - z) * nn + z * h0p
    h1_ref[...] = h1
    m1 = jnp.dot(h1, wg1_ref[...], preferred_element_type=jnp.float32)
    for f in range(F):
        m_refs[f][...] = m1[:, f * W:(f + 1) * W]


def _k3(hs, aggs, gh0, wiht, wg1, bih2):
    return pl.pallas_call(
        _k3_body,
        grid=(G,),
        in_specs=[pl.BlockSpec((R, 256), lambda r: (r, 0))] + [
            pl.BlockSpec((R, W), lambda r: (r, 0)) for _ in range(F)] + [
            pl.BlockSpec((R, 3 * OUT), lambda r: (r, 0)),
            pl.BlockSpec((OUT, 3 * OUT), lambda r: (0, 0)),
            pl.BlockSpec((OUT, OUT), lambda r: (0, 0)),
            pl.BlockSpec((1, 3 * OUT), lambda r: (0, 0)),
        ],
        out_specs=[pl.BlockSpec((R, OUT), lambda r: (r, 0))] + [
            pl.BlockSpec((R, W), lambda r: (r, 0)) for _ in range(F)],
        out_shape=[jax.ShapeDtypeStruct((N, OUT), jnp.float32)] + [
            jax.ShapeDtypeStruct((N, W), jnp.float32) for _ in range(F)],
    )(hs, *aggs, gh0, wiht, wg1, bih2)


def _k5_body(h1_ref, *rest):
    agg_refs = rest[:F]
    gh_ref, batch_ref, wiht_ref, bih_ref, t_ref, num_ref, den_ref = rest[F:]
    h1 = h1_ref[...]
    agg = jnp.concatenate([a[...] for a in agg_refs], axis=1)
    gi = bih_ref[...] + jnp.dot(agg, wiht_ref[...],
                                preferred_element_type=jnp.float32)
    gh = gh_ref[...]
    r = jax.nn.sigmoid(gi[:, :OUT] + gh[:, :OUT])
    z = jax.nn.sigmoid(gi[:, OUT:2 * OUT] + gh[:, OUT:2 * OUT])
    nn = jnp.tanh(gi[:, 2 * OUT:] + r * gh[:, 2 * OUT:])
    h2 = jax.nn.relu((1.0 - z) * nn + z * h1)
    e = jnp.exp(h2 * t_ref[0, 0])
    bb = batch_ref[0, 0, pl.ds(0, R)]
    ind = (lax.broadcasted_iota(jnp.int32, (B, R), 0) == bb[None, :]).astype(
        jnp.float32)
    nb = jnp.dot(ind, h2 * e, preferred_element_type=jnp.float32)
    db = jnp.dot(ind, e, preferred_element_type=jnp.float32)

    @pl.when(pl.program_id(0) == 0)
    def _():
        num_ref[...] = nb
        den_ref[...] = db

    @pl.when(pl.program_id(0) != 0)
    def _():
        num_ref[...] += nb
        den_ref[...] += db


def _k5(h1, aggs, gh1, batch3p, wiht, bih2, t128):
    return pl.pallas_call(
        _k5_body,
        grid=(G,),
        in_specs=[pl.BlockSpec((R, OUT), lambda r: (r, 0))] + [
            pl.BlockSpec((R, W), lambda r: (r, 0)) for _ in range(F)] + [
            pl.BlockSpec((R, 3 * OUT), lambda r: (r, 0)),
            pl.BlockSpec((1, 1, 512), lambda r: (r, 0, 0)),
            pl.BlockSpec((OUT, 3 * OUT), lambda r: (0, 0)),
            pl.BlockSpec((1, 3 * OUT), lambda r: (0, 0)),
            pl.BlockSpec((1, 128), lambda r: (0, 0)),
        ],
        out_specs=[pl.BlockSpec((B, OUT), lambda r: (0, 0)),
                   pl.BlockSpec((B, OUT), lambda r: (0, 0))],
        out_shape=[jax.ShapeDtypeStruct((B, OUT), jnp.float32),
                   jax.ShapeDtypeStruct((B, OUT), jnp.float32)],
    )(h1, *aggs, gh1, batch3p, wiht, bih2, t128)


def _k6_body(num_ref, den_ref, w1t_ref, b1_ref, wotp_ref, bop_ref, out_ref):
    pooled = num_ref[...] / (den_ref[...] + 1e-16)
    hid = jax.nn.relu(jnp.dot(pooled, w1t_ref[...],
                              preferred_element_type=jnp.float32) + b1_ref[...])
    lo = jnp.dot(hid, wotp_ref[...],
                 preferred_element_type=jnp.float32) + bop_ref[...]
    m = jnp.max(lo, axis=-1, keepdims=True)
    p = jnp.exp(lo - m)
    out_ref[...] = p / jnp.sum(p, axis=-1, keepdims=True)


def _k6(num, den, w1t, b1r, wotp, bop):
    return pl.pallas_call(
        _k6_body,
        out_shape=jax.ShapeDtypeStruct((B, 128), jnp.float32),
    )(num, den, w1t, b1r, wotp, bop)


def _sc_agg_body(*refs):
    m_refs = refs[:F]
    srcp_ref, dstp_ref, zsrc_ref = refs[F:F + 3]
    out_refs = refs[F + 3:F + 3 + F]
    src_g, dst_g, bufA, bufB, slab, gsemA, gsemB = refs[F + 3 + F:]
    bufs = (bufA, bufB)
    gsems = (gsemA, gsemB)
    cid = lax.axis_index("c")
    sid = lax.axis_index("s")

    for fc in range(F):
        m_ref = m_refs[fc]
        out_ref = out_refs[fc]

        @pl.when(cid == fc % 2)
        def _chunk():
            pltpu.sync_copy(zsrc_ref, slab.at[pl.ds(sid * ZROWS, ZROWS)])
            plsc.subcore_barrier()

            def _grp(g, c):
                base = sid * NBT + g * GS
                pltpu.sync_copy(srcp_ref.at[pl.ds(base, GS)], src_g)
                pltpu.sync_copy(dstp_ref.at[pl.ds(base, GS)], dst_g)
                pltpu.async_copy(m_ref.at[src_g.at[0]], bufs[0], gsems[0])
                for k in range(GS):
                    b = k % 2
                    if k + 1 < GS:
                        pltpu.async_copy(m_ref.at[src_g.at[k + 1]],
                                         bufs[1 - b], gsems[1 - b])
                    pltpu.make_async_copy(m_ref.at[src_g.at[k]],
                                          bufs[b], gsems[b]).wait()
                    pltpu.sync_copy(bufs[b], slab.at[dst_g.at[k]], add=True)
                return c

            lax.fori_loop(0, NGRP, _grp, 0)
            plsc.subcore_barrier()
            @pl.when(sid < 15)
            def _():
                pltpu.sync_copy(slab.at[pl.ds(sid * ZROWS, ZROWS)],
                                out_ref.at[pl.ds(sid * ZROWS, ZROWS)])

            @pl.when(sid == 15)
            def _():
                pltpu.sync_copy(slab.at[pl.ds(15 * ZROWS, FL_LAST)],
                                out_ref.at[pl.ds(15 * ZROWS, FL_LAST)])


def _sc_agg(m_list, srcp, dstp, zsrc):
    mesh = plsc.VectorSubcoreMesh(core_axis_name="c", subcore_axis_name="s")
    fn = functools.partial(
        pl.kernel,
        mesh=mesh,
        out_type=[jax.ShapeDtypeStruct((N, W), jnp.float32)
                  for _ in range(F)],
        scratch_types=[
            pltpu.VMEM((GS, 128), jnp.int32),
            pltpu.VMEM((GS, 128), jnp.int32),
            pltpu.VMEM((128, W), jnp.float32),
            pltpu.VMEM((128, W), jnp.float32),
            pltpu.VMEM_SHARED((SLAB, W), jnp.float32),
            pltpu.SemaphoreType.DMA,
            pltpu.SemaphoreType.DMA,
        ],
    )(_sc_agg_body)
    return fn(*m_list, srcp, dstp, zsrc)


def kernel(x, edge_index, batch, emb, ggc_w, W_ih, W_hh, b_ih, b_hh, t,
           W1, b1, Wo, bo):
    embp = jnp.pad(emb, ((0, 0), (0, 128 - EMB)))
    wg0p = jnp.pad(ggc_w[0][:159], ((0, 97), (0, 0)))
    whh0pt = jnp.pad(W_hh[:, :159].T, ((0, 97), (0, 0)))
    wiht = W_ih.T
    whht = W_hh.T
    wg1 = ggc_w[1]
    bih2 = b_ih.reshape(1, 3 * OUT)
    bhh2 = b_hh.reshape(1, 3 * OUT)
    t128 = jnp.broadcast_to(t.reshape(1, 1), (1, 128))
    w1t = W1.T
    b1r = b1.reshape(1, H1)
    wotp = jnp.pad(Wo.T, ((0, 0), (0, 128 - NC)))
    bop = jnp.concatenate(
        [bo, jnp.full((128 - NC,), -1e30, jnp.float32)]).reshape(1, 128)
    batch3p = jnp.pad(batch.reshape(G, 1, R), ((0, 0), (0, 0), (0, 512 - R)),
                      constant_values=B)

    src = edge_index[0]
    dst = edge_index[1]
    pad = EP - E
    srcp = jnp.concatenate(
        [src, jnp.zeros((pad,), jnp.int32)]).reshape(16 * NBT, 128)
    dstp = jnp.concatenate(
        [dst, jnp.full((pad,), N, jnp.int32)]).reshape(16 * NBT, 128)
    zsrc = jnp.zeros((ZROWS, W), jnp.float32)

    k1_out = _k1(x, embp, wg0p)
    hs, m0 = k1_out[0], k1_out[1:]
    agg0 = _sc_agg(m0, srcp, dstp, zsrc)
    gh0 = _k2(hs, whh0pt, bhh2)
    k3_out = _k3(hs, agg0, gh0, wiht, wg1, bih2)
    h1, m1 = k3_out[0], k3_out[1:]
    agg1 = _sc_agg(m1, srcp, dstp, zsrc)
    gh1 = _k2(h1, whht, bhh2)
    num, den = _k5(h1, agg1, gh1, batch3p, wiht, bih2, t128)
    outp = _k6(num, den, w1t, b1r, wotp, bop)
    return outp[:, :NC]


# speedup vs baseline: 1.0193x; 1.0193x over previous
import functools

import jax
import jax.numpy as jnp
from jax import lax
from jax.experimental import pallas as pl
from jax.experimental.pallas import tpu as pltpu
from jax.experimental.pallas import tpu_sc as plsc

N = 10000
E = 320000
IN = 128
EMB = 32
NT = 512
OUT = 1024
H1 = 512
NC = 10
B = 64

R = 400
G = N // R
F = 8
W = 128
NBT = 160
GS = 32
NGRP = NBT // GS
EP = 16 * NBT * 128
ZROWS = 632
SLAB = 16 * ZROWS
FL_LAST = N - 15 * ZROWS


def _k1_body(x_ref, embp_ref, wg0p_ref, hs_ref, *m_refs):
    xv = x_ref[...]
    ids = xv[:, 0:1].astype(jnp.int32)
    iota = lax.broadcasted_iota(jnp.int32, (R, NT), 1)
    oh = (ids == iota).astype(jnp.float32)
    te128 = jnp.dot(oh, embp_ref[...], preferred_element_type=jnp.float32)
    hs = jnp.concatenate(
        [te128[:, :EMB], xv[:, 1:], jnp.zeros((R, 97), jnp.float32)], axis=1)
    hs_ref[...] = hs
    m0 = jnp.dot(hs, wg0p_ref[...], preferred_element_type=jnp.float32)
    for f in range(F):
        m_refs[f][...] = m0[:, f * W:(f + 1) * W]


def _k1(x, embp, wg0p):
    return pl.pallas_call(
        _k1_body,
        grid=(G,),
        in_specs=[
            pl.BlockSpec((R, IN), lambda r: (r, 0)),
            pl.BlockSpec((NT, 128), lambda r: (0, 0)),
            pl.BlockSpec((256, OUT), lambda r: (0, 0)),
        ],
        out_specs=[pl.BlockSpec((R, 256), lambda r: (r, 0))] + [
            pl.BlockSpec((R, W), lambda r: (r, 0)) for _ in range(F)],
        out_shape=[jax.ShapeDtypeStruct((N, 256), jnp.float32)] + [
            jax.ShapeDtypeStruct((N, W), jnp.float32) for _ in range(F)],
    )(x, embp, wg0p)


def _k2_body(h_ref, w_ref, b_ref, gh_ref):
    gh_ref[...] = b_ref[...] + jnp.dot(h_ref[...], w_ref[...],
                                       preferred_element_type=jnp.float32)


def _k2(h, wt, b2):
    k = h.shape[1]
    return pl.pallas_call(
        _k2_body,
        grid=(G,),
        in_specs=[
            pl.BlockSpec((R, k), lambda r: (r, 0)),
            pl.BlockSpec((k, 3 * OUT), lambda r: (0, 0)),
            pl.BlockSpec((1, 3 * OUT), lambda r: (0, 0)),
        ],
        out_specs=pl.BlockSpec((R, 3 * OUT), lambda r: (r, 0)),
        out_shape=jax.ShapeDtypeStruct((N, 3 * OUT), jnp.float32),
    )(h, wt, b2)


def _k3_body(hs_ref, *rest):
    agg_refs = rest[:F]
    gh_ref, wiht_ref, wg1_ref, bih_ref, h1_ref = rest[F:F + 5]
    m_refs = rest[F + 5:]
    hs = hs_ref[...]
    agg = jnp.concatenate([a[...] for a in agg_refs], axis=1)
    gi = bih_ref[...] + jnp.dot(agg, wiht_ref[...],
                                preferred_element_type=jnp.float32)
    gh = gh_ref[...]
    r = jax.nn.sigmoid(gi[:, :OUT] + gh[:, :OUT])
    z = jax.nn.sigmoid(gi[:, OUT:2 * OUT] + gh[:, OUT:2 * OUT])
    nn = jnp.tanh(gi[:, 2 * OUT:] + r * gh[:, 2 * OUT:])
    h0p = jnp.concatenate([hs, jnp.zeros((R, OUT - 256), jnp.float32)], axis=1)
    h1 = (1.0 - z) * nn + z * h0p
    h1_ref[...] = h1
    m1 = jnp.dot(h1, wg1_ref[...], preferred_element_type=jnp.float32)
    for f in range(F):
        m_refs[f][...] = m1[:, f * W:(f + 1) * W]


def _k3(hs, aggs, gh0, wiht, wg1, bih2):
    return pl.pallas_call(
        _k3_body,
        grid=(G,),
        in_specs=[pl.BlockSpec((R, 256), lambda r: (r, 0))] + [
            pl.BlockSpec((R, W), lambda r: (r, 0)) for _ in range(F)] + [
            pl.BlockSpec((R, 3 * OUT), lambda r: (r, 0)),
            pl.BlockSpec((OUT, 3 * OUT), lambda r: (0, 0)),
            pl.BlockSpec((OUT, OUT), lambda r: (0, 0)),
            pl.BlockSpec((1, 3 * OUT), lambda r: (0, 0)),
        ],
        out_specs=[pl.BlockSpec((R, OUT), lambda r: (r, 0))] + [
            pl.BlockSpec((R, W), lambda r: (r, 0)) for _ in range(F)],
        out_shape=[jax.ShapeDtypeStruct((N, OUT), jnp.float32)] + [
            jax.ShapeDtypeStruct((N, W), jnp.float32) for _ in range(F)],
    )(hs, *aggs, gh0, wiht, wg1, bih2)


def _k5_body(h1_ref, *rest):
    agg_refs = rest[:F]
    gh_ref, batch_ref, wiht_ref, bih_ref, t_ref, num_ref, den_ref = rest[F:]
    h1 = h1_ref[...]
    agg = jnp.concatenate([a[...] for a in agg_refs], axis=1)
    gi = bih_ref[...] + jnp.dot(agg, wiht_ref[...],
                                preferred_element_type=jnp.float32)
    gh = gh_ref[...]
    r = jax.nn.sigmoid(gi[:, :OUT] + gh[:, :OUT])
    z = jax.nn.sigmoid(gi[:, OUT:2 * OUT] + gh[:, OUT:2 * OUT])
    nn = jnp.tanh(gi[:, 2 * OUT:] + r * gh[:, 2 * OUT:])
    h2 = jax.nn.relu((1.0 - z) * nn + z * h1)
    e = jnp.exp(h2 * t_ref[0, 0])
    bb = batch_ref[0, 0, pl.ds(0, R)]
    ind = (lax.broadcasted_iota(jnp.int32, (B, R), 0) == bb[None, :]).astype(
        jnp.float32)
    nb = jnp.dot(ind, h2 * e, preferred_element_type=jnp.float32)
    db = jnp.dot(ind, e, preferred_element_type=jnp.float32)

    @pl.when(pl.program_id(0) == 0)
    def _():
        num_ref[...] = nb
        den_ref[...] = db

    @pl.when(pl.program_id(0) != 0)
    def _():
        num_ref[...] += nb
        den_ref[...] += db


def _k5(h1, aggs, gh1, batch3p, wiht, bih2, t128):
    return pl.pallas_call(
        _k5_body,
        grid=(G,),
        in_specs=[pl.BlockSpec((R, OUT), lambda r: (r, 0))] + [
            pl.BlockSpec((R, W), lambda r: (r, 0)) for _ in range(F)] + [
            pl.BlockSpec((R, 3 * OUT), lambda r: (r, 0)),
            pl.BlockSpec((1, 1, 512), lambda r: (r, 0, 0)),
            pl.BlockSpec((OUT, 3 * OUT), lambda r: (0, 0)),
            pl.BlockSpec((1, 3 * OUT), lambda r: (0, 0)),
            pl.BlockSpec((1, 128), lambda r: (0, 0)),
        ],
        out_specs=[pl.BlockSpec((B, OUT), lambda r: (0, 0)),
                   pl.BlockSpec((B, OUT), lambda r: (0, 0))],
        out_shape=[jax.ShapeDtypeStruct((B, OUT), jnp.float32),
                   jax.ShapeDtypeStruct((B, OUT), jnp.float32)],
    )(h1, *aggs, gh1, batch3p, wiht, bih2, t128)


def _k6_body(num_ref, den_ref, w1t_ref, b1_ref, wotp_ref, bop_ref, out_ref):
    pooled = num_ref[...] / (den_ref[...] + 1e-16)
    hid = jax.nn.relu(jnp.dot(pooled, w1t_ref[...],
                              preferred_element_type=jnp.float32) + b1_ref[...])
    lo = jnp.dot(hid, wotp_ref[...],
                 preferred_element_type=jnp.float32) + bop_ref[...]
    m = jnp.max(lo, axis=-1, keepdims=True)
    p = jnp.exp(lo - m)
    out_ref[...] = p / jnp.sum(p, axis=-1, keepdims=True)


def _k6(num, den, w1t, b1r, wotp, bop):
    return pl.pallas_call(
        _k6_body,
        out_shape=jax.ShapeDtypeStruct((B, 128), jnp.float32),
    )(num, den, w1t, b1r, wotp, bop)


def _sc_agg_body(*refs):
    m_refs = refs[:F]
    srcp_ref, dstp_ref, zsrc_ref = refs[F:F + 3]
    out_refs = refs[F + 3:F + 3 + F]
    src_g, dst_g, bufA, bufB, slab, gsemA, gsemB = refs[F + 3 + F:]
    bufs = (bufA, bufB)
    gsems = (gsemA, gsemB)
    cid = lax.axis_index("c")
    sid = lax.axis_index("s")

    for fc in range(F):
        m_ref = m_refs[fc]
        out_ref = out_refs[fc]

        @pl.when(cid == fc % 2)
        def _chunk():
            pltpu.sync_copy(zsrc_ref, slab.at[pl.ds(sid * ZROWS, ZROWS)])
            plsc.subcore_barrier()

            def _grp(g, c):
                base = sid * NBT + g * GS
                pltpu.sync_copy(srcp_ref.at[pl.ds(base, GS)], src_g)
                pltpu.sync_copy(dstp_ref.at[pl.ds(base, GS)], dst_g)
                pltpu.async_copy(m_ref.at[src_g.at[0]], bufs[0], gsems[0])
                for k in range(GS):
                    b = k % 2
                    if k + 1 < GS:
                        pltpu.async_copy(m_ref.at[src_g.at[k + 1]],
                                         bufs[1 - b], gsems[1 - b])
                    pltpu.make_async_copy(m_ref.at[src_g.at[k]],
                                          bufs[b], gsems[b]).wait()
                    pltpu.sync_copy(bufs[b], slab.at[dst_g.at[k]], add=True)
                return c

            lax.fori_loop(0, NGRP, _grp, 0)
            plsc.subcore_barrier()
            @pl.when(sid < 15)
            def _():
                pltpu.sync_copy(slab.at[pl.ds(sid * ZROWS, ZROWS)],
                                out_ref.at[pl.ds(sid * ZROWS, ZROWS)])

            @pl.when(sid == 15)
            def _():
                pltpu.sync_copy(slab.at[pl.ds(15 * ZROWS, FL_LAST)],
                                out_ref.at[pl.ds(15 * ZROWS, FL_LAST)])


def _sc_agg(m_list, srcp, dstp, zsrc):
    mesh = plsc.VectorSubcoreMesh(core_axis_name="c", subcore_axis_name="s")
    fn = functools.partial(
        pl.kernel,
        mesh=mesh,
        out_type=[jax.ShapeDtypeStruct((N, W), jnp.float32)
                  for _ in range(F)],
        scratch_types=[
            pltpu.VMEM((GS, 128), jnp.int32),
            pltpu.VMEM((GS, 128), jnp.int32),
            pltpu.VMEM((128, W), jnp.float32),
            pltpu.VMEM((128, W), jnp.float32),
            pltpu.VMEM_SHARED((SLAB, W), jnp.float32),
            pltpu.SemaphoreType.DMA,
            pltpu.SemaphoreType.DMA,
        ],
    )(_sc_agg_body)
    return fn(*m_list, srcp, dstp, zsrc)


def kernel(x, edge_index, batch, emb, ggc_w, W_ih, W_hh, b_ih, b_hh, t,
           W1, b1, Wo, bo):
    embp = jnp.pad(emb, ((0, 0), (0, 128 - EMB)))
    wg0p = jnp.pad(ggc_w[0][:159], ((0, 97), (0, 0)))
    whh0pt = jnp.pad(W_hh[:, :159].T, ((0, 97), (0, 0)))
    wiht = W_ih.T
    whht = W_hh.T
    wg1 = ggc_w[1]
    bih2 = b_ih.reshape(1, 3 * OUT)
    bhh2 = b_hh.reshape(1, 3 * OUT)
    t128 = jnp.broadcast_to(t.reshape(1, 1), (1, 128))
    w1t = W1.T
    b1r = b1.reshape(1, H1)
    wotp = jnp.pad(Wo.T, ((0, 0), (0, 128 - NC)))
    bop = jnp.concatenate(
        [bo, jnp.full((128 - NC,), -1e30, jnp.float32)]).reshape(1, 128)
    batch3p = jnp.pad(batch.reshape(G, 1, R), ((0, 0), (0, 0), (0, 512 - R)),
                      constant_values=B)

    src = edge_index[0]
    dst = edge_index[1]
    pad = EP - E
    srcp = jnp.concatenate(
        [src, jnp.zeros((pad,), jnp.int32)]).reshape(16 * NBT, 128)
    dstp = jnp.concatenate(
        [dst, jnp.full((pad,), N, jnp.int32)]).reshape(16 * NBT, 128)
    zsrc = jnp.zeros((ZROWS, W), jnp.float32)

    k1_out = _k1(x, embp, wg0p)
    hs, m0 = k1_out[0], k1_out[1:]
    agg0 = _sc_agg(m0, srcp, dstp, zsrc)
    gh0 = _k2(hs, whh0pt, bhh2)
    k3_out = _k3(hs, agg0, gh0, wiht, wg1, bih2)
    h1, m1 = k3_out[0], k3_out[1:]
    agg1 = _sc_agg(m1, srcp, dstp, zsrc)
    gh1 = _k2(h1, whht, bhh2)
    num, den = _k5(h1, agg1, gh1, batch3p, wiht, bih2, t128)
    outp = _k6(num, den, w1t, b1r, wotp, bop)
    return outp[:, :NC]


# speedup vs baseline: 1.0230x; 1.0036x over previous
import functools

import jax
import jax.numpy as jnp
from jax import lax
from jax.experimental import pallas as pl
from jax.experimental.pallas import tpu as pltpu
from jax.experimental.pallas import tpu_sc as plsc

N = 10000
E = 320000
IN = 128
EMB = 32
NT = 512
OUT = 1024
H1 = 512
NC = 10
B = 64

R = 400
G = N // R
F = 8
W = 128
NBT = 160
GS = 40
NGRP = NBT // GS
EP = 16 * NBT * 128
ZROWS = 632
SLAB = 16 * ZROWS
FL_LAST = N - 15 * ZROWS


def _k1_body(x_ref, embp_ref, wg0p_ref, hs_ref, *m_refs):
    xv = x_ref[...]
    ids = xv[:, 0:1].astype(jnp.int32)
    iota = lax.broadcasted_iota(jnp.int32, (R, NT), 1)
    oh = (ids == iota).astype(jnp.float32)
    te128 = jnp.dot(oh, embp_ref[...], preferred_element_type=jnp.float32)
    hs = jnp.concatenate(
        [te128[:, :EMB], xv[:, 1:], jnp.zeros((R, 97), jnp.float32)], axis=1)
    hs_ref[...] = hs
    m0 = jnp.dot(hs, wg0p_ref[...], preferred_element_type=jnp.float32)
    for f in range(F):
        m_refs[f][...] = m0[:, f * W:(f + 1) * W]


def _k1(x, embp, wg0p):
    return pl.pallas_call(
        _k1_body,
        grid=(G,),
        in_specs=[
            pl.BlockSpec((R, IN), lambda r: (r, 0)),
            pl.BlockSpec((NT, 128), lambda r: (0, 0)),
            pl.BlockSpec((256, OUT), lambda r: (0, 0)),
        ],
        out_specs=[pl.BlockSpec((R, 256), lambda r: (r, 0))] + [
            pl.BlockSpec((R, W), lambda r: (r, 0)) for _ in range(F)],
        out_shape=[jax.ShapeDtypeStruct((N, 256), jnp.float32)] + [
            jax.ShapeDtypeStruct((N, W), jnp.float32) for _ in range(F)],
    )(x, embp, wg0p)


def _k2_body(h_ref, w_ref, b_ref, gh_ref):
    gh_ref[...] = b_ref[...] + jnp.dot(h_ref[...], w_ref[...],
                                       preferred_element_type=jnp.float32)


def _k2(h, wt, b2):
    k = h.shape[1]
    return pl.pallas_call(
        _k2_body,
        grid=(G,),
        in_specs=[
            pl.BlockSpec((R, k), lambda r: (r, 0)),
            pl.BlockSpec((k, 3 * OUT), lambda r: (0, 0)),
            pl.BlockSpec((1, 3 * OUT), lambda r: (0, 0)),
        ],
        out_specs=pl.BlockSpec((R, 3 * OUT), lambda r: (r, 0)),
        out_shape=jax.ShapeDtypeStruct((N, 3 * OUT), jnp.float32),
    )(h, wt, b2)


def _k3_body(hs_ref, *rest):
    agg_refs = rest[:F]
    gh_ref, wiht_ref, wg1_ref, bih_ref, h1_ref = rest[F:F + 5]
    m_refs = rest[F + 5:]
    hs = hs_ref[...]
    agg = jnp.concatenate([a[...] for a in agg_refs], axis=1)
    gi = bih_ref[...] + jnp.dot(agg, wiht_ref[...],
                                preferred_element_type=jnp.float32)
    gh = gh_ref[...]
    r = jax.nn.sigmoid(gi[:, :OUT] + gh[:, :OUT])
    z = jax.nn.sigmoid(gi[:, OUT:2 * OUT] + gh[:, OUT:2 * OUT])
    nn = jnp.tanh(gi[:, 2 * OUT:] + r * gh[:, 2 * OUT:])
    h0p = jnp.concatenate([hs, jnp.zeros((R, OUT - 256), jnp.float32)], axis=1)
    h1 = (1.0 - z) * nn + z * h0p
    h1_ref[...] = h1
    m1 = jnp.dot(h1, wg1_ref[...], preferred_element_type=jnp.float32)
    for f in range(F):
        m_refs[f][...] = m1[:, f * W:(f + 1) * W]


def _k3(hs, aggs, gh0, wiht, wg1, bih2):
    return pl.pallas_call(
        _k3_body,
        grid=(G,),
        in_specs=[pl.BlockSpec((R, 256), lambda r: (r, 0))] + [
            pl.BlockSpec((R, W), lambda r: (r, 0)) for _ in range(F)] + [
            pl.BlockSpec((R, 3 * OUT), lambda r: (r, 0)),
            pl.BlockSpec((OUT, 3 * OUT), lambda r: (0, 0)),
            pl.BlockSpec((OUT, OUT), lambda r: (0, 0)),
            pl.BlockSpec((1, 3 * OUT), lambda r: (0, 0)),
        ],
        out_specs=[pl.BlockSpec((R, OUT), lambda r: (r, 0))] + [
            pl.BlockSpec((R, W), lambda r: (r, 0)) for _ in range(F)],
        out_shape=[jax.ShapeDtypeStruct((N, OUT), jnp.float32)] + [
            jax.ShapeDtypeStruct((N, W), jnp.float32) for _ in range(F)],
    )(hs, *aggs, gh0, wiht, wg1, bih2)


def _k5_body(h1_ref, *rest):
    agg_refs = rest[:F]
    gh_ref, batch_ref, wiht_ref, bih_ref, t_ref, num_ref, den_ref = rest[F:]
    h1 = h1_ref[...]
    agg = jnp.concatenate([a[...] for a in agg_refs], axis=1)
    gi = bih_ref[...] + jnp.dot(agg, wiht_ref[...],
                                preferred_element_type=jnp.float32)
    gh = gh_ref[...]
    r = jax.nn.sigmoid(gi[:, :OUT] + gh[:, :OUT])
    z = jax.nn.sigmoid(gi[:, OUT:2 * OUT] + gh[:, OUT:2 * OUT])
    nn = jnp.tanh(gi[:, 2 * OUT:] + r * gh[:, 2 * OUT:])
    h2 = jax.nn.relu((1.0 - z) * nn + z * h1)
    e = jnp.exp(h2 * t_ref[0, 0])
    bb = batch_ref[0, 0, pl.ds(0, R)]
    ind = (lax.broadcasted_iota(jnp.int32, (B, R), 0) == bb[None, :]).astype(
        jnp.float32)
    nb = jnp.dot(ind, h2 * e, preferred_element_type=jnp.float32)
    db = jnp.dot(ind, e, preferred_element_type=jnp.float32)

    @pl.when(pl.program_id(0) == 0)
    def _():
        num_ref[...] = nb
        den_ref[...] = db

    @pl.when(pl.program_id(0) != 0)
    def _():
        num_ref[...] += nb
        den_ref[...] += db


def _k5(h1, aggs, gh1, batch3p, wiht, bih2, t128):
    return pl.pallas_call(
        _k5_body,
        grid=(G,),
        in_specs=[pl.BlockSpec((R, OUT), lambda r: (r, 0))] + [
            pl.BlockSpec((R, W), lambda r: (r, 0)) for _ in range(F)] + [
            pl.BlockSpec((R, 3 * OUT), lambda r: (r, 0)),
            pl.BlockSpec((1, 1, 512), lambda r: (r, 0, 0)),
            pl.BlockSpec((OUT, 3 * OUT), lambda r: (0, 0)),
            pl.BlockSpec((1, 3 * OUT), lambda r: (0, 0)),
            pl.BlockSpec((1, 128), lambda r: (0, 0)),
        ],
        out_specs=[pl.BlockSpec((B, OUT), lambda r: (0, 0)),
                   pl.BlockSpec((B, OUT), lambda r: (0, 0))],
        out_shape=[jax.ShapeDtypeStruct((B, OUT), jnp.float32),
                   jax.ShapeDtypeStruct((B, OUT), jnp.float32)],
    )(h1, *aggs, gh1, batch3p, wiht, bih2, t128)


def _k6_body(num_ref, den_ref, w1t_ref, b1_ref, wotp_ref, bop_ref, out_ref):
    pooled = num_ref[...] / (den_ref[...] + 1e-16)
    hid = jax.nn.relu(jnp.dot(pooled, w1t_ref[...],
                              preferred_element_type=jnp.float32) + b1_ref[...])
    lo = jnp.dot(hid, wotp_ref[...],
                 preferred_element_type=jnp.float32) + bop_ref[...]
    m = jnp.max(lo, axis=-1, keepdims=True)
    p = jnp.exp(lo - m)
    out_ref[...] = p / jnp.sum(p, axis=-1, keepdims=True)


def _k6(num, den, w1t, b1r, wotp, bop):
    return pl.pallas_call(
        _k6_body,
        out_shape=jax.ShapeDtypeStruct((B, 128), jnp.float32),
    )(num, den, w1t, b1r, wotp, bop)


def _sc_agg_body(*refs):
    m_refs = refs[:F]
    srcp_ref, dstp_ref, zsrc_ref = refs[F:F + 3]
    out_refs = refs[F + 3:F + 3 + F]
    src_g, dst_g, bufA, bufB, slab, gsemA, gsemB = refs[F + 3 + F:]
    bufs = (bufA, bufB)
    gsems = (gsemA, gsemB)
    cid = lax.axis_index("c")
    sid = lax.axis_index("s")

    for fc in range(F):
        m_ref = m_refs[fc]
        out_ref = out_refs[fc]

        @pl.when(cid == fc % 2)
        def _chunk():
            pltpu.sync_copy(zsrc_ref, slab.at[pl.ds(sid * ZROWS, ZROWS)])
            plsc.subcore_barrier()

            def _grp(g, c):
                base = sid * NBT + g * GS
                pltpu.sync_copy(srcp_ref.at[pl.ds(base, GS)], src_g)
                pltpu.sync_copy(dstp_ref.at[pl.ds(base, GS)], dst_g)
                pltpu.async_copy(m_ref.at[src_g.at[0]], bufs[0], gsems[0])
                for k in range(GS):
                    b = k % 2
                    if k + 1 < GS:
                        pltpu.async_copy(m_ref.at[src_g.at[k + 1]],
                                         bufs[1 - b], gsems[1 - b])
                    pltpu.make_async_copy(m_ref.at[src_g.at[k]],
                                          bufs[b], gsems[b]).wait()
                    pltpu.sync_copy(bufs[b], slab.at[dst_g.at[k]], add=True)
                return c

            lax.fori_loop(0, NGRP, _grp, 0)
            plsc.subcore_barrier()
            @pl.when(sid < 15)
            def _():
                pltpu.sync_copy(slab.at[pl.ds(sid * ZROWS, ZROWS)],
                                out_ref.at[pl.ds(sid * ZROWS, ZROWS)])

            @pl.when(sid == 15)
            def _():
                pltpu.sync_copy(slab.at[pl.ds(15 * ZROWS, FL_LAST)],
                                out_ref.at[pl.ds(15 * ZROWS, FL_LAST)])


def _sc_agg(m_list, srcp, dstp, zsrc):
    mesh = plsc.VectorSubcoreMesh(core_axis_name="c", subcore_axis_name="s")
    fn = functools.partial(
        pl.kernel,
        mesh=mesh,
        out_type=[jax.ShapeDtypeStruct((N, W), jnp.float32)
                  for _ in range(F)],
        scratch_types=[
            pltpu.VMEM((GS, 128), jnp.int32),
            pltpu.VMEM((GS, 128), jnp.int32),
            pltpu.VMEM((128, W), jnp.float32),
            pltpu.VMEM((128, W), jnp.float32),
            pltpu.VMEM_SHARED((SLAB, W), jnp.float32),
            pltpu.SemaphoreType.DMA,
            pltpu.SemaphoreType.DMA,
        ],
    )(_sc_agg_body)
    return fn(*m_list, srcp, dstp, zsrc)


def kernel(x, edge_index, batch, emb, ggc_w, W_ih, W_hh, b_ih, b_hh, t,
           W1, b1, Wo, bo):
    embp = jnp.pad(emb, ((0, 0), (0, 128 - EMB)))
    wg0p = jnp.pad(ggc_w[0][:159], ((0, 97), (0, 0)))
    whh0pt = jnp.pad(W_hh[:, :159].T, ((0, 97), (0, 0)))
    wiht = W_ih.T
    whht = W_hh.T
    wg1 = ggc_w[1]
    bih2 = b_ih.reshape(1, 3 * OUT)
    bhh2 = b_hh.reshape(1, 3 * OUT)
    t128 = jnp.broadcast_to(t.reshape(1, 1), (1, 128))
    w1t = W1.T
    b1r = b1.reshape(1, H1)
    wotp = jnp.pad(Wo.T, ((0, 0), (0, 128 - NC)))
    bop = jnp.concatenate(
        [bo, jnp.full((128 - NC,), -1e30, jnp.float32)]).reshape(1, 128)
    batch3p = jnp.pad(batch.reshape(G, 1, R), ((0, 0), (0, 0), (0, 512 - R)),
                      constant_values=B)

    src = edge_index[0]
    dst = edge_index[1]
    pad = EP - E
    srcp = jnp.concatenate(
        [src, jnp.zeros((pad,), jnp.int32)]).reshape(16 * NBT, 128)
    dstp = jnp.concatenate(
        [dst, jnp.full((pad,), N, jnp.int32)]).reshape(16 * NBT, 128)
    zsrc = jnp.zeros((ZROWS, W), jnp.float32)

    k1_out = _k1(x, embp, wg0p)
    hs, m0 = k1_out[0], k1_out[1:]
    agg0 = _sc_agg(m0, srcp, dstp, zsrc)
    gh0 = _k2(hs, whh0pt, bhh2)
    k3_out = _k3(hs, agg0, gh0, wiht, wg1, bih2)
    h1, m1 = k3_out[0], k3_out[1:]
    agg1 = _sc_agg(m1, srcp, dstp, zsrc)
    gh1 = _k2(h1, whht, bhh2)
    num, den = _k5(h1, agg1, gh1, batch3p, wiht, bih2, t128)
    outp = _k6(num, den, w1t, b1r, wotp, bop)
    return outp[:, :NC]
